# baseline (speedup 1.0000x reference)
import functools

import jax
import jax.numpy as jnp
from jax import lax
from jax.experimental import pallas as pl
from jax.experimental.pallas import tpu as pltpu

_BT = 256
_BR = 512


def _router_rank_body(x_ref, rw_ref, w_ref, idx_ref, rank_ref, counts_ref,
                      xbf_ref, lt_ref, carry_ref):
    c = pl.program_id(0)
    bt = x_ref.shape[0]
    E = rw_ref.shape[0]

    @pl.when(c == 0)
    def _init():
        ii = lax.broadcasted_iota(jnp.int32, (bt, bt), 0)
        jj = lax.broadcasted_iota(jnp.int32, (bt, bt), 1)
        lt_ref[...] = (jj < ii).astype(jnp.float32)
        carry_ref[...] = jnp.zeros_like(carry_ref)

    x = x_ref[...]
    xbf_ref[...] = x.astype(jnp.bfloat16)
    logits = lax.dot_general(
        x, rw_ref[...], (((1,), (1,)), ((), ())),
        preferred_element_type=jnp.float32,
    )
    s = jax.nn.sigmoid(logits)
    eio = lax.broadcasted_iota(jnp.int32, (bt, E), 1)
    m1 = jnp.max(s, axis=1, keepdims=True)
    i1 = jnp.min(jnp.where(s == m1, eio, E), axis=1, keepdims=True)
    s2 = jnp.where(eio == i1, jnp.float32(-1.0), s)
    m2 = jnp.max(s2, axis=1, keepdims=True)
    i2 = jnp.min(jnp.where(s2 == m2, eio, E), axis=1, keepdims=True)
    denom = m1 + m2 + jnp.float32(1e-20)
    w_ref[...] = jnp.concatenate([m1, m2], axis=1) / denom
    idx_ref[...] = jnp.concatenate([i1, i2], axis=1)

    oh0 = (eio == i1).astype(jnp.float32)
    oh1 = (eio == i2).astype(jnp.float32)
    both = oh0 + oh1
    pfx = lax.dot_general(
        lt_ref[...], both, (((1,), (0,)), ((), ())),
        preferred_element_type=jnp.float32,
    )
    base = carry_ref[0:1, :]
    r = pfx + base
    rank0 = jnp.sum(oh0 * r, axis=1, keepdims=True)
    rank1 = jnp.sum(oh1 * r, axis=1, keepdims=True)
    rank_ref[...] = jnp.concatenate([rank0, rank1], axis=1).astype(jnp.int32)
    newc = base + jnp.sum(both, axis=0, keepdims=True)
    carry_ref[0:1, :] = newc
    counts_ref[...] = newc.astype(jnp.int32)


def _moe_body(be_ref, ba_ref, xs_ref, g_ref, u_ref, d_ref, y_ref):
    b = pl.program_id(0)

    @pl.when(ba_ref[b] == 1)
    def _():
        xb = xs_ref[...]
        t1 = lax.dot_general(
            xb, g_ref[0], (((1,), (1,)), ((), ())),
            preferred_element_type=jnp.float32,
        )
        t2 = lax.dot_general(
            xb, u_ref[0], (((1,), (1,)), ((), ())),
            preferred_element_type=jnp.float32,
        )
        h = t1 * jax.nn.sigmoid(t1) * t2
        o = lax.dot_general(
            h, d_ref[0], (((1,), (1,)), ((), ())),
            preferred_element_type=jnp.float32,
        )
        y_ref[...] = o.astype(y_ref.dtype)

    @pl.when(ba_ref[b] == 0)
    def _():
        y_ref[...] = jnp.zeros_like(y_ref)


@functools.partial(jax.jit, static_argnames=())
def kernel(hidden_states, router_w, gate_w, up_w, down_w):
    orig_shape = hidden_states.shape
    H = orig_shape[-1]
    x = hidden_states.reshape(-1, H)
    T = x.shape[0]
    E, F, _ = gate_w.shape
    K = 2
    N = T * K
    n_pad = N + E * _BT
    nb = n_pad // _BT

    w2, idx2, rank2, counts, x_bf = pl.pallas_call(
        _router_rank_body,
        grid=(T // _BR,),
        in_specs=[
            pl.BlockSpec((_BR, H), lambda i: (i, 0)),
            pl.BlockSpec((E, H), lambda i: (0, 0)),
        ],
        out_specs=[
            pl.BlockSpec((_BR, K), lambda i: (i, 0)),
            pl.BlockSpec((_BR, K), lambda i: (i, 0)),
            pl.BlockSpec((_BR, K), lambda i: (i, 0)),
            pl.BlockSpec((1, E), lambda i: (0, 0)),
            pl.BlockSpec((_BR, H), lambda i: (i, 0)),
        ],
        out_shape=[
            jax.ShapeDtypeStruct((T, K), jnp.float32),
            jax.ShapeDtypeStruct((T, K), jnp.int32),
            jax.ShapeDtypeStruct((T, K), jnp.int32),
            jax.ShapeDtypeStruct((1, E), jnp.int32),
            jax.ShapeDtypeStruct((T, H), jnp.bfloat16),
        ],
        scratch_shapes=[
            pltpu.VMEM((_BR, _BR), jnp.float32),
            pltpu.VMEM((8, E), jnp.float32),
        ],
    )(x, router_w)

    cnt = counts[0]
    padded = ((cnt + _BT - 1) // _BT) * _BT
    pad_cum = jnp.concatenate(
        [jnp.zeros((1,), jnp.int32), jnp.cumsum(padded)[:-1].astype(jnp.int32)]
    )
    e_flat = idx2.reshape(-1)
    seg_base = jnp.sum(
        (e_flat[:, None] == jnp.arange(E)[None, :]) * pad_cum[None, :], axis=1
    ).astype(jnp.int32)
    dst = seg_base + rank2.reshape(-1)

    block_starts = jnp.arange(nb, dtype=jnp.int32) * _BT
    cmp = (block_starts[:, None] >= pad_cum[None, :]).astype(jnp.int32)
    block_expert = jnp.sum(cmp, axis=1) - 1
    be_oh = block_expert[:, None] == jnp.arange(E)[None, :]
    pc_sel = jnp.sum(be_oh * pad_cum[None, :], axis=1).astype(jnp.int32)
    c_sel = jnp.sum(be_oh * cnt[None, :], axis=1).astype(jnp.int32)
    block_active = (block_starts - pc_sel < c_sel).astype(jnp.int32)

    tok_of = jnp.arange(N, dtype=jnp.int32) // K
    src_tok = jnp.zeros((n_pad,), jnp.int32).at[dst].set(tok_of)
    xs = jnp.take(x_bf, src_tok, axis=0)

    y = pl.pallas_call(
        _moe_body,
        grid_spec=pltpu.PrefetchScalarGridSpec(
            num_scalar_prefetch=2,
            grid=(nb,),
            in_specs=[
                pl.BlockSpec((_BT, H), lambda b, be, ba: (b, 0)),
                pl.BlockSpec((1, F, H), lambda b, be, ba: (be[b], 0, 0)),
                pl.BlockSpec((1, F, H), lambda b, be, ba: (be[b], 0, 0)),
                pl.BlockSpec((1, H, F), lambda b, be, ba: (be[b], 0, 0)),
            ],
            out_specs=pl.BlockSpec((_BT, H), lambda b, be, ba: (b, 0)),
        ),
        out_shape=jax.ShapeDtypeStruct((n_pad, H), jnp.bfloat16),
        compiler_params=pltpu.CompilerParams(
            dimension_semantics=("arbitrary",),
        ),
    )(block_expert, block_active, xs, gate_w, up_w, down_w)

    dst2 = dst.reshape(T, K)
    y0 = jnp.take(y, dst2[:, 0], axis=0).astype(jnp.float32)
    y1 = jnp.take(y, dst2[:, 1], axis=0).astype(jnp.float32)
    final = w2[:, 0:1] * y0 + w2[:, 1:2] * y1
    return final.reshape(orig_shape)
